# baseline (speedup 1.0000x reference)
import functools

import jax
import jax.numpy as jnp
from jax import lax
from jax.experimental import pallas as pl
from jax.experimental.pallas import tpu as pltpu
from jax.experimental.pallas import tpu_sc as plsc

N = 10000
E = 320000
D = 128
BN_EPS = 1e-5

NC = 2
NS = 16
NW = NC * NS
EPW = E // NW
CH = 80
NCHUNK = EPW // CH
BLKC = 25
NBLK = NCHUNK // BLKC
N_PAD = 10240
ROWS_PT = N_PAD // NS

def _aggregate_body(feat_hbm, ei_hbm, out_hbm,
                    src_v, dst_v, rows_a, rows_b, acc_sh,
                    sem_i, sem_a, sem_b, sem_sa, sem_sb):
    c = lax.axis_index("c")
    s = lax.axis_index("s")
    wid = s * NC + c

    cp_src = pltpu.async_copy(ei_hbm.at[0, wid, 0], src_v.at[0], sem_i)
    cp_dst = pltpu.async_copy(ei_hbm.at[1, wid, 0], dst_v.at[0], sem_i)

    zeros16 = jnp.zeros((16,), jnp.float32)

    def _zrow(r, carry):
        for j in range(D // 16):
            rows_a[r, pl.ds(j * 16, 16)] = zeros16
        return carry

    lax.fori_loop(0, CH, _zrow, 0)

    def _zcopy(i, carry):
        pltpu.sync_copy(rows_a, acc_sh.at[pl.ds(s * ROWS_PT + i * CH, CH)])
        return carry

    lax.fori_loop(0, ROWS_PT // CH, _zcopy, 0)
    cp_src.wait()
    cp_dst.wait()
    plsc.subcore_barrier()

    def _block(b, carry):
        p = lax.rem(b, 2)

        @pl.when(b < NBLK - 1)
        def _prefetch():
            pltpu.async_copy(ei_hbm.at[0, wid, b + 1], src_v.at[1 - p], sem_i)
            pltpu.async_copy(ei_hbm.at[1, wid, b + 1], dst_v.at[1 - p], sem_i)

        pltpu.async_copy(feat_hbm.at[src_v.at[p, 0]], rows_a, sem_a)
        pltpu.async_copy(feat_hbm.at[src_v.at[p, 1]], rows_b, sem_b)

        def _pair(g, cc):
            i = 2 * g
            pltpu.make_async_copy(feat_hbm.at[src_v.at[p, i]], rows_a,
                                  sem_a).wait()
            pltpu.async_copy(rows_a, acc_sh.at[dst_v.at[p, i]], sem_sa,
                             add=True)
            pltpu.make_async_copy(feat_hbm.at[src_v.at[p, i + 1]], rows_b,
                                  sem_b).wait()
            pltpu.async_copy(rows_b, acc_sh.at[dst_v.at[p, i + 1]], sem_sb,
                             add=True)
            pltpu.make_async_copy(rows_a, acc_sh.at[dst_v.at[p, i]],
                                  sem_sa).wait()
            pltpu.async_copy(feat_hbm.at[src_v.at[p, i + 2]], rows_a, sem_a)
            pltpu.make_async_copy(rows_b, acc_sh.at[dst_v.at[p, i + 1]],
                                  sem_sb).wait()

            @pl.when(i + 3 < BLKC)
            def _refill_b():
                pltpu.async_copy(feat_hbm.at[src_v.at[p, i + 3]], rows_b,
                                 sem_b)

            return cc

        lax.fori_loop(0, (BLKC - 1) // 2, _pair, 0)
        pltpu.make_async_copy(feat_hbm.at[src_v.at[p, BLKC - 1]], rows_a,
                              sem_a).wait()
        pltpu.sync_copy(rows_a, acc_sh.at[dst_v.at[p, BLKC - 1]], add=True)

        @pl.when(b < NBLK - 1)
        def _wait_prefetch():
            pltpu.make_async_copy(ei_hbm.at[0, wid, b + 1], src_v.at[1 - p],
                                  sem_i).wait()
            pltpu.make_async_copy(ei_hbm.at[1, wid, b + 1], dst_v.at[1 - p],
                                  sem_i).wait()

        return carry

    lax.fori_loop(0, NBLK, _block, 0)
    plsc.subcore_barrier()

    pltpu.sync_copy(acc_sh.at[pl.ds(s * ROWS_PT, ROWS_PT)],
                    out_hbm.at[c, pl.ds(s * ROWS_PT, ROWS_PT)])


@functools.cache
def _make_aggregate():
    mesh = plsc.VectorSubcoreMesh(core_axis_name="c", subcore_axis_name="s",
                                  num_cores=NC, num_subcores=NS)
    return pl.kernel(
        _aggregate_body,
        out_type=jax.ShapeDtypeStruct((NC, N_PAD, D), jnp.float32),
        mesh=mesh,
        scratch_types=[
            pltpu.VMEM((2, BLKC, CH), jnp.int32),
            pltpu.VMEM((2, BLKC, CH), jnp.int32),
            pltpu.VMEM((CH, D), jnp.float32),
            pltpu.VMEM((CH, D), jnp.float32),
            pltpu.VMEM_SHARED((N_PAD, D), jnp.float32),
            pltpu.SemaphoreType.DMA,
            pltpu.SemaphoreType.DMA,
            pltpu.SemaphoreType.DMA,
            pltpu.SemaphoreType.DMA,
            pltpu.SemaphoreType.DMA,
        ],
    )


def _mlp_body(feat_ref, parts_ref, eps_ref, w1_ref, b1_ref, gamma_ref,
              beta_ref, w2_ref, b2_ref, out_ref):
    out_h = parts_ref[0, :N] + parts_ref[1, :N]
    h = (1.0 + eps_ref[0, 0]) * feat_ref[...] + out_h
    a = jnp.dot(h, w1_ref[...], preferred_element_type=jnp.float32) \
        + b1_ref[...]
    mean = jnp.mean(a, axis=0, keepdims=True)
    var = jnp.mean(jnp.square(a - mean), axis=0, keepdims=True)
    a = (a - mean) * lax.rsqrt(var + BN_EPS) * gamma_ref[...] + beta_ref[...]
    a = jnp.maximum(a, 0.0)
    out_ref[...] = jnp.dot(a, w2_ref[...],
                           preferred_element_type=jnp.float32) + b2_ref[...]


_mlp = pl.pallas_call(
    _mlp_body,
    out_shape=jax.ShapeDtypeStruct((N, D), jnp.float32),
)


def kernel(feature, edge_index, eps, W1, b1, gamma, beta, W2, b2):
    ei = edge_index.reshape(2, NW, NBLK, BLKC, CH)
    parts = _make_aggregate()(feature, ei)
    eps2 = jnp.reshape(eps, (1, 1)).astype(jnp.float32)
    return _mlp(feature, parts, eps2, W1,
                b1.reshape(1, 2 * D), gamma.reshape(1, 2 * D),
                beta.reshape(1, 2 * D), W2, b2.reshape(1, D))


# speedup vs baseline: 1.1740x; 1.1740x over previous
import functools

import jax
import jax.numpy as jnp
from jax import lax
from jax.experimental import pallas as pl
from jax.experimental.pallas import tpu as pltpu
from jax.experimental.pallas import tpu_sc as plsc

N = 10000
E = 320000
D = 128
BN_EPS = 1e-5

NC = 2
NS = 16
NW = NC * NS
EPW = E // NW
CH = 80
NCHUNK = EPW // CH
BLKC = 25
NBLK = NCHUNK // BLKC
N_PAD = 10240
ROWS_PT = N_PAD // NS

def _aggregate_body(feat_hbm, ei_hbm, out_hbm,
                    src_v, dst_v, rows_a, rows_b, rows_c, acc_sh,
                    sem_i, sem_a, sem_b, sem_c, sem_sa, sem_sb, sem_sc,
                    sem_z):
    c = lax.axis_index("c")
    s = lax.axis_index("s")
    wid = s * NC + c

    cp_src = pltpu.async_copy(ei_hbm.at[0, wid, 0], src_v.at[0], sem_i)
    cp_dst = pltpu.async_copy(ei_hbm.at[1, wid, 0], dst_v.at[0], sem_i)

    zeros16 = jnp.zeros((16,), jnp.float32)

    def _zrow(r, carry):
        for j in range(D // 16):
            rows_a[r, pl.ds(j * 16, 16)] = zeros16
        return carry

    lax.fori_loop(0, CH, _zrow, 0)

    def _zfire(i, carry):
        pltpu.async_copy(rows_a, acc_sh.at[pl.ds(s * ROWS_PT + i * CH, CH)],
                         sem_z)
        return carry

    lax.fori_loop(0, ROWS_PT // CH, _zfire, 0)

    def _zdrain(i, carry):
        pltpu.make_async_copy(
            rows_a, acc_sh.at[pl.ds(s * ROWS_PT + i * CH, CH)], sem_z).wait()
        return carry

    lax.fori_loop(0, ROWS_PT // CH, _zdrain, 0)
    cp_src.wait()
    cp_dst.wait()
    plsc.subcore_barrier()

    def _gather(p, j, buf, sem):
        pltpu.async_copy(feat_hbm.at[src_v.at[p, j]], buf, sem)

    def _gwait(p, j, buf, sem):
        pltpu.make_async_copy(feat_hbm.at[src_v.at[p, j]], buf, sem).wait()

    def _sfire(p, j, buf, sem):
        pltpu.async_copy(buf, acc_sh.at[dst_v.at[p, j]], sem, add=True)

    def _swait(p, j, buf, sem):
        pltpu.make_async_copy(buf, acc_sh.at[dst_v.at[p, j]], sem).wait()

    bufs = ((rows_a, sem_a, sem_sa), (rows_b, sem_b, sem_sb),
            (rows_c, sem_c, sem_sc))

    def _block(b, carry):
        p = lax.rem(b, 2)

        @pl.when(b < NBLK - 1)
        def _prefetch():
            pltpu.async_copy(ei_hbm.at[0, wid, b + 1], src_v.at[1 - p], sem_i)
            pltpu.async_copy(ei_hbm.at[1, wid, b + 1], dst_v.at[1 - p], sem_i)

        for k in range(3):
            _gather(p, k, bufs[k][0], bufs[k][1])

        def _trio(t, cc):
            i = 3 * t
            for k in range(3):
                _gwait(p, i + k, bufs[k][0], bufs[k][1])
                _sfire(p, i + k, bufs[k][0], bufs[k][2])
            for k in range(3):
                _swait(p, i + k, bufs[k][0], bufs[k][2])
                _gather(p, i + 3 + k, bufs[k][0], bufs[k][1])
            return cc

        lax.fori_loop(0, (BLKC - 4) // 3, _trio, 0)

        i = BLKC - 4
        for k in range(3):
            _gwait(p, i + k, bufs[k][0], bufs[k][1])
            _sfire(p, i + k, bufs[k][0], bufs[k][2])
        _swait(p, i, bufs[0][0], bufs[0][2])
        _gather(p, BLKC - 1, bufs[0][0], bufs[0][1])
        _swait(p, i + 1, bufs[1][0], bufs[1][2])
        _swait(p, i + 2, bufs[2][0], bufs[2][2])
        _gwait(p, BLKC - 1, bufs[0][0], bufs[0][1])
        _sfire(p, BLKC - 1, bufs[0][0], bufs[0][2])
        _swait(p, BLKC - 1, bufs[0][0], bufs[0][2])

        @pl.when(b < NBLK - 1)
        def _wait_prefetch():
            pltpu.make_async_copy(ei_hbm.at[0, wid, b + 1], src_v.at[1 - p],
                                  sem_i).wait()
            pltpu.make_async_copy(ei_hbm.at[1, wid, b + 1], dst_v.at[1 - p],
                                  sem_i).wait()

        return carry

    lax.fori_loop(0, NBLK, _block, 0)
    plsc.subcore_barrier()

    pltpu.sync_copy(acc_sh.at[pl.ds(s * ROWS_PT, ROWS_PT)],
                    out_hbm.at[c, pl.ds(s * ROWS_PT, ROWS_PT)])


@functools.cache
def _make_aggregate():
    mesh = plsc.VectorSubcoreMesh(core_axis_name="c", subcore_axis_name="s",
                                  num_cores=NC, num_subcores=NS)
    return pl.kernel(
        _aggregate_body,
        out_type=jax.ShapeDtypeStruct((NC, N_PAD, D), jnp.float32),
        mesh=mesh,
        scratch_types=[
            pltpu.VMEM((2, BLKC, CH), jnp.int32),
            pltpu.VMEM((2, BLKC, CH), jnp.int32),
            pltpu.VMEM((CH, D), jnp.float32),
            pltpu.VMEM((CH, D), jnp.float32),
            pltpu.VMEM((CH, D), jnp.float32),
            pltpu.VMEM_SHARED((N_PAD, D), jnp.float32),
        ] + [pltpu.SemaphoreType.DMA] * 8,
    )


def _mlp_body(feat_ref, parts_ref, eps_ref, w1_ref, b1_ref, gamma_ref,
              beta_ref, w2_ref, b2_ref, out_ref):
    out_h = parts_ref[0, :N] + parts_ref[1, :N]
    h = (1.0 + eps_ref[0, 0]) * feat_ref[...] + out_h
    a = jnp.dot(h, w1_ref[...], preferred_element_type=jnp.float32) \
        + b1_ref[...]
    mean = jnp.mean(a, axis=0, keepdims=True)
    var = jnp.mean(jnp.square(a - mean), axis=0, keepdims=True)
    a = (a - mean) * lax.rsqrt(var + BN_EPS) * gamma_ref[...] + beta_ref[...]
    a = jnp.maximum(a, 0.0)
    out_ref[...] = jnp.dot(a, w2_ref[...],
                           preferred_element_type=jnp.float32) + b2_ref[...]


_mlp = pl.pallas_call(
    _mlp_body,
    out_shape=jax.ShapeDtypeStruct((N, D), jnp.float32),
)


def kernel(feature, edge_index, eps, W1, b1, gamma, beta, W2, b2):
    ei = edge_index.reshape(2, NW, NBLK, BLKC, CH)
    parts = _make_aggregate()(feature, ei)
    eps2 = jnp.reshape(eps, (1, 1)).astype(jnp.float32)
    return _mlp(feature, parts, eps2, W1,
                b1.reshape(1, 2 * D), gamma.reshape(1, 2 * D),
                beta.reshape(1, 2 * D), W2, b2.reshape(1, D))


# speedup vs baseline: 1.2156x; 1.0354x over previous
import functools

import jax
import jax.numpy as jnp
from jax import lax
from jax.experimental import pallas as pl
from jax.experimental.pallas import tpu as pltpu
from jax.experimental.pallas import tpu_sc as plsc

N = 10000
E = 320000
D = 128
BN_EPS = 1e-5

NC = 2
NS = 16
NW = NC * NS
EPW = E // NW
CH = 80
NCHUNK = EPW // CH
BLKC = 25
NBLK = NCHUNK // BLKC
N_PAD = 10240
ROWS_PT = N_PAD // NS

def _aggregate_body(feat_hbm, ei_hbm, out_hbm,
                    src_v, dst_v, rows_a, rows_b, acc_sh,
                    sem_i, sem_a, sem_b, sem_z):
    c = lax.axis_index("c")
    s = lax.axis_index("s")
    wid = s * NC + c

    cp_src = pltpu.async_copy(ei_hbm.at[0, wid, 0], src_v.at[0], sem_i)
    cp_dst = pltpu.async_copy(ei_hbm.at[1, wid, 0], dst_v.at[0], sem_i)

    zeros16 = jnp.zeros((16,), jnp.float32)

    def _zrow(r, carry):
        for j in range(D // 16):
            rows_a[r, pl.ds(j * 16, 16)] = zeros16
        return carry

    lax.fori_loop(0, CH, _zrow, 0)

    def _zfire(i, carry):
        pltpu.async_copy(rows_a, acc_sh.at[pl.ds(s * ROWS_PT + i * CH, CH)],
                         sem_z)
        return carry

    lax.fori_loop(0, ROWS_PT // CH, _zfire, 0)

    def _zdrain(i, carry):
        pltpu.make_async_copy(
            rows_a, acc_sh.at[pl.ds(s * ROWS_PT + i * CH, CH)], sem_z).wait()
        return carry

    lax.fori_loop(0, ROWS_PT // CH, _zdrain, 0)
    cp_src.wait()
    cp_dst.wait()
    plsc.subcore_barrier()

    def _block(b, carry):
        p = lax.rem(b, 2)

        @pl.when(b < NBLK - 1)
        def _prefetch():
            pltpu.async_copy(ei_hbm.at[0, wid, b + 1], src_v.at[1 - p], sem_i)
            pltpu.async_copy(ei_hbm.at[1, wid, b + 1], dst_v.at[1 - p], sem_i)

        pltpu.async_copy(feat_hbm.at[src_v.at[p, 0]], rows_a, sem_a)

        def _pair(g, cc):
            i = 2 * g
            pltpu.async_copy(feat_hbm.at[src_v.at[p, i + 1]], rows_b, sem_b)
            pltpu.make_async_copy(feat_hbm.at[src_v.at[p, i]], rows_a,
                                  sem_a).wait()
            pltpu.sync_copy(rows_a, acc_sh.at[dst_v.at[p, i]], add=True)
            pltpu.async_copy(feat_hbm.at[src_v.at[p, i + 2]], rows_a, sem_a)
            pltpu.make_async_copy(feat_hbm.at[src_v.at[p, i + 1]], rows_b,
                                  sem_b).wait()
            pltpu.sync_copy(rows_b, acc_sh.at[dst_v.at[p, i + 1]], add=True)
            return cc

        lax.fori_loop(0, (BLKC - 1) // 2, _pair, 0)
        pltpu.make_async_copy(feat_hbm.at[src_v.at[p, BLKC - 1]], rows_a,
                              sem_a).wait()
        pltpu.sync_copy(rows_a, acc_sh.at[dst_v.at[p, BLKC - 1]], add=True)

        @pl.when(b < NBLK - 1)
        def _wait_prefetch():
            pltpu.make_async_copy(ei_hbm.at[0, wid, b + 1], src_v.at[1 - p],
                                  sem_i).wait()
            pltpu.make_async_copy(ei_hbm.at[1, wid, b + 1], dst_v.at[1 - p],
                                  sem_i).wait()

        return carry

    lax.fori_loop(0, NBLK, _block, 0)
    plsc.subcore_barrier()

    pltpu.sync_copy(acc_sh.at[pl.ds(s * ROWS_PT, ROWS_PT)],
                    out_hbm.at[c, pl.ds(s * ROWS_PT, ROWS_PT)])


@functools.cache
def _make_aggregate():
    mesh = plsc.VectorSubcoreMesh(core_axis_name="c", subcore_axis_name="s",
                                  num_cores=NC, num_subcores=NS)
    return pl.kernel(
        _aggregate_body,
        out_type=jax.ShapeDtypeStruct((NC, N_PAD, D), jnp.float32),
        mesh=mesh,
        scratch_types=[
            pltpu.VMEM((2, BLKC, CH), jnp.int32),
            pltpu.VMEM((2, BLKC, CH), jnp.int32),
            pltpu.VMEM((CH, D), jnp.float32),
            pltpu.VMEM((CH, D), jnp.float32),
            pltpu.VMEM_SHARED((N_PAD, D), jnp.float32),
            pltpu.SemaphoreType.DMA,
            pltpu.SemaphoreType.DMA,
            pltpu.SemaphoreType.DMA,
            pltpu.SemaphoreType.DMA,
        ],
    )


def _mlp_body(feat_ref, parts_ref, eps_ref, w1_ref, b1_ref, gamma_ref,
              beta_ref, w2_ref, b2_ref, out_ref):
    out_h = parts_ref[0, :N] + parts_ref[1, :N]
    h = (1.0 + eps_ref[0, 0]) * feat_ref[...] + out_h
    a = jnp.dot(h, w1_ref[...], preferred_element_type=jnp.float32) \
        + b1_ref[...]
    mean = jnp.mean(a, axis=0, keepdims=True)
    var = jnp.mean(jnp.square(a - mean), axis=0, keepdims=True)
    a = (a - mean) * lax.rsqrt(var + BN_EPS) * gamma_ref[...] + beta_ref[...]
    a = jnp.maximum(a, 0.0)
    out_ref[...] = jnp.dot(a, w2_ref[...],
                           preferred_element_type=jnp.float32) + b2_ref[...]


_mlp = pl.pallas_call(
    _mlp_body,
    out_shape=jax.ShapeDtypeStruct((N, D), jnp.float32),
)


def kernel(feature, edge_index, eps, W1, b1, gamma, beta, W2, b2):
    ei = edge_index.reshape(2, NW, NBLK, BLKC, CH)
    parts = _make_aggregate()(feature, ei)
    eps2 = jnp.reshape(eps, (1, 1)).astype(jnp.float32)
    return _mlp(feature, parts, eps2, W1,
                b1.reshape(1, 2 * D), gamma.reshape(1, 2 * D),
                beta.reshape(1, 2 * D), W2, b2.reshape(1, D))
